# baseline (speedup 1.0000x reference)
import functools

import jax
import jax.numpy as jnp
from jax import lax
from jax.experimental import pallas as pl
from jax.experimental.pallas import tpu as pltpu
from jax.experimental.pallas import tpu_sc as plsc

_info = plsc.get_sparse_core_info()
_NC = _info.num_cores
_NS = _info.num_subcores
_NW = _NC * _NS


def _iota16():
  return jax.lax.iota(jnp.int32, 16)


@functools.lru_cache(maxsize=None)
def _make_table_linear(vocab, dim):
  nblk = vocab // 128
  rem = vocab - nblk * 128
  per_w = nblk // _NW
  extra = nblk - per_w * _NW
  mesh = plsc.VectorSubcoreMesh(core_axis_name="c", subcore_axis_name="s")

  @functools.partial(
      pl.kernel,
      mesh=mesh,
      out_type=jax.ShapeDtypeStruct((vocab * dim,), jnp.float32),
      compiler_params=pltpu.CompilerParams(
          use_tc_tiling_on_sc=True, needs_layout_passes=False
      ),
      scratch_types=[
          pltpu.VMEM((dim, 128), jnp.float32),
          pltpu.VMEM((dim, 128), jnp.float32),
          pltpu.VMEM((128 * dim,), jnp.float32),
          pltpu.VMEM((128 * dim,), jnp.float32),
      ]
      + [pltpu.SemaphoreType.DMA] * 4,
  )
  def detile(tt_hbm, tail_hbm, out_hbm, tile_v0, tile_v1, lin_v0, lin_v1,
             *sems):
    tile_v = (tile_v0, tile_v1)
    lin_v = (lin_v0, lin_v1)
    isem = sems[:2]
    osem = sems[2:]
    w = lax.axis_index("s") * _NC + lax.axis_index("c")
    iota = _iota16()
    perm = [(iota + k) & 15 for k in range(16)]
    vk = [(((iota + k) & 15) * dim) + iota for k in range(16)]

    def c0_of(j):
      return (j * _NW + w) * 128

    def start_in(p, j):
      pltpu.async_copy(
          tt_hbm.at[pl.ds(0, dim), pl.ds(c0_of(j), 128)],
          tile_v[p],
          isem[p],
      )

    def wait_in(p):
      pltpu.make_async_copy(
          tt_hbm.at[pl.ds(0, dim), pl.ds(0, 128)], tile_v[p], isem[p]
      ).wait()

    def start_out(p, j):
      pltpu.async_copy(
          lin_v[p], out_hbm.at[pl.ds(c0_of(j) * dim, 128 * dim)], osem[p]
      )

    def wait_out(p):
      pltpu.make_async_copy(
          out_hbm.at[pl.ds(0, 128 * dim)], lin_v[p], osem[p]
      ).wait()

    iota_dim = iota * dim

    def transpose_block(p, width):
      def row(d, carry):
        for cl0 in range(0, width, 16):
          x = tile_v[p][d, pl.ds(cl0, 16)]
          plsc.store_scatter(lin_v[p], [iota_dim + (cl0 * dim + d)], x)
        return carry

      lax.fori_loop(0, dim, row, 0)

    start_in(0, 0)
    start_in(1, 1)

    def body(j2, carry):
      for p in range(2):
        j = 2 * j2 + p
        wait_in(p)

        @pl.when(j2 >= 1)
        def _():
          wait_out(p)

        transpose_block(p, 128)

        @pl.when(j < per_w - 2)
        def _():
          start_in(p, j + 2)

        start_out(p, j)
      return carry

    lax.fori_loop(0, per_w // 2, body, 0)
    wait_out(0)
    wait_out(1)

    if extra:

      @pl.when(w < extra)
      def _():
        start_in(0, per_w)
        wait_in(0)
        transpose_block(0, 128)
        start_out(0, per_w)
        wait_out(0)

    if rem:

      @pl.when(w == _NW - 1)
      def _():
        nt = rem * dim
        pltpu.sync_copy(tail_hbm, lin_v1.at[pl.ds(0, nt)])
        pltpu.sync_copy(
            lin_v1.at[pl.ds(0, nt)],
            out_hbm.at[pl.ds(nblk * 128 * dim, nt)],
        )

  return detile


@functools.lru_cache(maxsize=None)
def _make_gather_pack(vocab, seq, n_bt, dim):
  st_n = seq // 8
  mesh = plsc.VectorSubcoreMesh(core_axis_name="c", subcore_axis_name="s")

  nbuf = 2

  @functools.partial(
      pl.kernel,
      mesh=mesh,
      out_type=jax.ShapeDtypeStruct((seq, dim // 8, n_bt, 1024),
                                    jnp.float32),
      compiler_params=pltpu.CompilerParams(
          use_tc_tiling_on_sc=False, needs_layout_passes=False
      ),
      scratch_types=[
          pltpu.VMEM((st_n, 8, 128), jnp.int32),
          pltpu.VMEM((nbuf, 128, dim), jnp.float32),
          pltpu.VMEM((nbuf, dim // 8, 1024), jnp.float32),
      ]
      + [pltpu.SemaphoreType.DMA] * (2 * nbuf),
  )
  def gather_pack(table_hbm, inp4_hbm, out_hbm, idx_v, rows_v, t_v, *sems):
    gsem = sems[:nbuf]
    osem = sems[nbuf:]
    w = lax.axis_index("s") * _NC + lax.axis_index("c")
    pltpu.sync_copy(inp4_hbm.at[pl.ds(0, st_n), w], idx_v)

    iota = _iota16()
    iota_g = [(iota >> 3) + 2 * h for h in range(dim // 16)]
    iota_j = (iota & 7) * 128

    def start_gather(p, s):
      pltpu.async_copy(
          table_hbm.at[idx_v.at[s // 8, s % 8]], rows_v.at[p], gsem[p]
      )

    def wait_gather(p):
      pltpu.make_async_copy(
          table_hbm.at[pl.ds(0, 128)], rows_v.at[p], gsem[p]
      ).wait()

    def start_write(p, s):
      pltpu.async_copy(
          t_v.at[p], out_hbm.at[s, pl.ds(0, dim // 8), w], osem[p]
      )

    def wait_write(p):
      pltpu.make_async_copy(
          out_hbm.at[0, pl.ds(0, dim // 8), 0], t_v.at[p], osem[p]
      ).wait()

    for p in range(nbuf):
      start_gather(p, p)

    def body(s4, carry):
      for p in range(nbuf):
        s = nbuf * s4 + p
        wait_gather(p)

        @pl.when(s4 >= 1)
        def _():
          wait_write(p)

        def tok_group(ti, carry):
          t0 = ti * 8
          for tt in range(8):
            t = t0 + tt
            jvec = iota_j + t
            for h in range(dim // 16):
              x = rows_v[p, t, pl.ds(16 * h, 16)]
              plsc.store_scatter(t_v.at[p], [iota_g[h], jvec], x)
          return carry

        lax.fori_loop(0, 16, tok_group, 0)

        @pl.when(s4 < seq // nbuf - 1)
        def _():
          start_gather(p, s + nbuf)

        start_write(p, s)
      return carry

    lax.fori_loop(0, seq // nbuf, body, 0)
    for p in range(nbuf):
      wait_write(p)

  return gather_pack


@functools.lru_cache(maxsize=None)
def _make_gather(vocab, dim, n, chunk, nbuf):
  n_per_w = n // _NW
  n_chunks = n_per_w // chunk
  n_groups = n_chunks // nbuf
  mesh = plsc.VectorSubcoreMesh(core_axis_name="c", subcore_axis_name="s")

  @functools.partial(
      pl.kernel,
      mesh=mesh,
      out_type=jax.ShapeDtypeStruct((n, dim), jnp.float32),
      compiler_params=pltpu.CompilerParams(use_tc_tiling_on_sc=False),
      scratch_types=[
          pltpu.VMEM((n_per_w,), jnp.int32),
          pltpu.VMEM((nbuf, chunk, dim), jnp.float32),
      ]
      + [pltpu.SemaphoreType.DMA] * (2 * nbuf),
  )
  def gather_kernel(table_hbm, idx_hbm, out_hbm, idx_v, rows_v, *sems):
    gsem = sems[:nbuf]
    osem = sems[nbuf:]
    wid = lax.axis_index("s") * _NC + lax.axis_index("c")
    base = wid * n_per_w
    pltpu.sync_copy(idx_hbm.at[pl.ds(base, n_per_w)], idx_v)

    def start_gather(b, c):
      pltpu.async_copy(
          table_hbm.at[idx_v.at[pl.ds(c * chunk, chunk)]],
          rows_v.at[b],
          gsem[b],
      )

    def wait_gather(b):
      pltpu.make_async_copy(
          table_hbm.at[pl.ds(0, chunk)], rows_v.at[b], gsem[b]
      ).wait()

    def start_out(b, c):
      pltpu.async_copy(
          rows_v.at[b], out_hbm.at[pl.ds(base + c * chunk, chunk)], osem[b]
      )

    def wait_out(b):
      pltpu.make_async_copy(
          out_hbm.at[pl.ds(base, chunk)], rows_v.at[b], osem[b]
      ).wait()

    for b in range(nbuf):
      start_gather(b, b)

    def group_body(g, carry):
      c0 = g * nbuf
      for b in range(nbuf):
        wait_gather(b)
        start_out(b, c0 + b)
      for b in range(nbuf):
        wait_out(b)
        start_gather(b, c0 + nbuf + b)
      return carry

    lax.fori_loop(0, n_groups - 1, group_body, 0)

    c0 = (n_groups - 1) * nbuf
    for b in range(nbuf):
      wait_gather(b)
      start_out(b, c0 + b)
    for b in range(nbuf):
      wait_out(b)

  return gather_kernel


def kernel(inp, table):
  b, s = inp.shape
  vocab, dim = table.shape
  n = b * s
  inp = inp.astype(jnp.int32)

  per_w = (vocab // 128) // _NW
  fast = (
      b == 128 * _NW
      and s % 8 == 0
      and dim == 32
      and vocab % 16 == 0
      and per_w >= 2
      and per_w % 2 == 0
  )
  if fast:
    n_bt = b // 128
    nblk = vocab // 128
    rem = vocab - nblk * 128
    if rem:
      tail = table[nblk * 128 :].reshape(rem * dim)
    else:
      tail = jnp.zeros((128,), jnp.float32)
    table_lin = _make_table_linear(vocab, dim)(table.T, tail)
    table_rm = table_lin.reshape(vocab, dim)
    inp4 = inp.T.reshape(s // 8, 8, n_bt, 128).transpose(0, 2, 1, 3)
    out4 = _make_gather_pack(vocab, s, n_bt, dim)(table_rm, inp4)
    out5 = out4.reshape(s, dim // 8, n_bt, 8, 128)
    return out5.transpose(2, 4, 0, 1, 3).reshape(b, s, dim)

  flat = inp.reshape(n)
  chunk, nbuf = 640, 5
  pad = (-n) % (_NW * chunk * nbuf)
  if pad:
    flat = jnp.concatenate([flat, jnp.zeros((pad,), jnp.int32)])
  rows = _make_gather(vocab, dim, n + pad, chunk, nbuf)(table, flat)
  if pad:
    rows = rows[:n]
  return rows.reshape(b, s, dim)


# speedup vs baseline: 2.2933x; 2.2933x over previous
import functools

import jax
import jax.numpy as jnp
from jax import lax
from jax.experimental import pallas as pl
from jax.experimental.pallas import tpu as pltpu
from jax.experimental.pallas import tpu_sc as plsc

_info = plsc.get_sparse_core_info()
_NC = _info.num_cores
_NS = _info.num_subcores
_NW = _NC * _NS


def _iota16():
  return jax.lax.iota(jnp.int32, 16)


@functools.lru_cache(maxsize=None)
def _make_table_linear(vocab, dim):
  nblk = vocab // 128
  rem = vocab - nblk * 128
  per_w = nblk // _NW
  extra = nblk - per_w * _NW
  mesh = plsc.VectorSubcoreMesh(core_axis_name="c", subcore_axis_name="s")

  @functools.partial(
      pl.kernel,
      mesh=mesh,
      out_type=jax.ShapeDtypeStruct((vocab * dim,), jnp.float32),
      compiler_params=pltpu.CompilerParams(
          use_tc_tiling_on_sc=True, needs_layout_passes=False
      ),
      scratch_types=[
          pltpu.VMEM((dim, 128), jnp.float32),
          pltpu.VMEM((dim, 128), jnp.float32),
          pltpu.VMEM((128 * dim,), jnp.float32),
          pltpu.VMEM((128 * dim,), jnp.float32),
      ]
      + [pltpu.SemaphoreType.DMA] * 4,
  )
  def detile(tt_hbm, tail_hbm, out_hbm, tile_v0, tile_v1, lin_v0, lin_v1,
             *sems):
    tile_v = (tile_v0, tile_v1)
    lin_v = (lin_v0, lin_v1)
    isem = sems[:2]
    osem = sems[2:]
    w = lax.axis_index("s") * _NC + lax.axis_index("c")
    iota = _iota16()
    perm = [(iota + k) & 15 for k in range(16)]
    vk = [(((iota + k) & 15) * dim) + iota for k in range(16)]

    def c0_of(j):
      return (j * _NW + w) * 128

    def start_in(p, j):
      pltpu.async_copy(
          tt_hbm.at[pl.ds(0, dim), pl.ds(c0_of(j), 128)],
          tile_v[p],
          isem[p],
      )

    def wait_in(p):
      pltpu.make_async_copy(
          tt_hbm.at[pl.ds(0, dim), pl.ds(0, 128)], tile_v[p], isem[p]
      ).wait()

    def start_out(p, j):
      pltpu.async_copy(
          lin_v[p], out_hbm.at[pl.ds(c0_of(j) * dim, 128 * dim)], osem[p]
      )

    def wait_out(p):
      pltpu.make_async_copy(
          out_hbm.at[pl.ds(0, 128 * dim)], lin_v[p], osem[p]
      ).wait()

    def transpose_block(p, width):
      def strip(ci, carry):
        cl0 = ci * 16
        for d0 in range(0, dim, 16):
          base = cl0 * dim + d0
          for k in range(16):
            x = plsc.load_gather(
                tile_v[p], [iota + d0, perm[k] + cl0]
            )
            plsc.store_scatter(lin_v[p], [vk[k] + base], x)
        return carry

      lax.fori_loop(0, width // 16, strip, 0)

    start_in(0, 0)
    start_in(1, 1)

    def body(j2, carry):
      for p in range(2):
        j = 2 * j2 + p
        wait_in(p)

        @pl.when(j2 >= 1)
        def _():
          wait_out(p)

        transpose_block(p, 128)

        @pl.when(j < per_w - 2)
        def _():
          start_in(p, j + 2)

        start_out(p, j)
      return carry

    lax.fori_loop(0, per_w // 2, body, 0)
    wait_out(0)
    wait_out(1)

    if extra:

      @pl.when(w < extra)
      def _():
        start_in(0, per_w)
        wait_in(0)
        transpose_block(0, 128)
        start_out(0, per_w)
        wait_out(0)

    if rem:

      @pl.when(w == _NW - 1)
      def _():
        nt = rem * dim
        pltpu.sync_copy(tail_hbm, lin_v1.at[pl.ds(0, nt)])
        pltpu.sync_copy(
            lin_v1.at[pl.ds(0, nt)],
            out_hbm.at[pl.ds(nblk * 128 * dim, nt)],
        )

  return detile


@functools.lru_cache(maxsize=None)
def _make_gather_pack(vocab, seq, n_bt, dim):
  st_n = seq // 8
  mesh = plsc.VectorSubcoreMesh(core_axis_name="c", subcore_axis_name="s")

  nbuf = 2

  @functools.partial(
      pl.kernel,
      mesh=mesh,
      out_type=jax.ShapeDtypeStruct((seq, dim // 8, n_bt, 1024),
                                    jnp.float32),
      compiler_params=pltpu.CompilerParams(
          use_tc_tiling_on_sc=False, needs_layout_passes=False
      ),
      scratch_types=[
          pltpu.VMEM((st_n, 8, 128), jnp.int32),
          pltpu.VMEM((nbuf, 128, dim), jnp.float32),
          pltpu.VMEM((nbuf, dim // 8, 1024), jnp.float32),
      ]
      + [pltpu.SemaphoreType.DMA] * (2 * nbuf),
  )
  def gather_pack(table_hbm, inp4_hbm, out_hbm, idx_v, rows_v, t_v, *sems):
    gsem = sems[:nbuf]
    osem = sems[nbuf:]
    w = lax.axis_index("s") * _NC + lax.axis_index("c")
    pltpu.sync_copy(inp4_hbm.at[pl.ds(0, st_n), w], idx_v)

    iota = _iota16()
    perm = [(iota + k) & 15 for k in range(16)]
    gk = [((iota + k) & 15) >> 3 for k in range(16)]
    jk = [(((iota + k) & 7) * 128) + iota for k in range(16)]

    def start_gather(p, s):
      pltpu.async_copy(
          table_hbm.at[idx_v.at[s // 8, s % 8]], rows_v.at[p], gsem[p]
      )

    def wait_gather(p):
      pltpu.make_async_copy(
          table_hbm.at[pl.ds(0, 128)], rows_v.at[p], gsem[p]
      ).wait()

    def start_write(p, s):
      pltpu.async_copy(
          t_v.at[p], out_hbm.at[s, pl.ds(0, dim // 8), w], osem[p]
      )

    def wait_write(p):
      pltpu.make_async_copy(
          out_hbm.at[0, pl.ds(0, dim // 8), 0], t_v.at[p], osem[p]
      ).wait()

    for p in range(nbuf):
      start_gather(p, p)

    def body(s4, carry):
      for p in range(nbuf):
        s = nbuf * s4 + p
        wait_gather(p)

        @pl.when(s4 >= 1)
        def _():
          wait_write(p)

        def strip(ti, carry):
          t0 = ti * 16
          rvec = iota + t0
          for k in range(16):
            jvec = jk[k] + t0
            for d0 in range(0, dim, 16):
              x = plsc.load_gather(
                  rows_v.at[p], [rvec, perm[k] + d0]
              )
              plsc.store_scatter(
                  t_v.at[p], [gk[k] + (d0 >> 3), jvec], x
              )
          return carry

        lax.fori_loop(0, 8, strip, 0)

        @pl.when(s4 < seq // nbuf - 1)
        def _():
          start_gather(p, s + nbuf)

        start_write(p, s)
      return carry

    lax.fori_loop(0, seq // nbuf, body, 0)
    for p in range(nbuf):
      wait_write(p)

  return gather_pack


@functools.lru_cache(maxsize=None)
def _make_gather(vocab, dim, n, chunk, nbuf):
  n_per_w = n // _NW
  n_chunks = n_per_w // chunk
  n_groups = n_chunks // nbuf
  mesh = plsc.VectorSubcoreMesh(core_axis_name="c", subcore_axis_name="s")

  @functools.partial(
      pl.kernel,
      mesh=mesh,
      out_type=jax.ShapeDtypeStruct((n, dim), jnp.float32),
      compiler_params=pltpu.CompilerParams(use_tc_tiling_on_sc=False),
      scratch_types=[
          pltpu.VMEM((n_per_w,), jnp.int32),
          pltpu.VMEM((nbuf, chunk, dim), jnp.float32),
      ]
      + [pltpu.SemaphoreType.DMA] * (2 * nbuf),
  )
  def gather_kernel(table_hbm, idx_hbm, out_hbm, idx_v, rows_v, *sems):
    gsem = sems[:nbuf]
    osem = sems[nbuf:]
    wid = lax.axis_index("s") * _NC + lax.axis_index("c")
    base = wid * n_per_w
    pltpu.sync_copy(idx_hbm.at[pl.ds(base, n_per_w)], idx_v)

    def start_gather(b, c):
      pltpu.async_copy(
          table_hbm.at[idx_v.at[pl.ds(c * chunk, chunk)]],
          rows_v.at[b],
          gsem[b],
      )

    def wait_gather(b):
      pltpu.make_async_copy(
          table_hbm.at[pl.ds(0, chunk)], rows_v.at[b], gsem[b]
      ).wait()

    def start_out(b, c):
      pltpu.async_copy(
          rows_v.at[b], out_hbm.at[pl.ds(base + c * chunk, chunk)], osem[b]
      )

    def wait_out(b):
      pltpu.make_async_copy(
          out_hbm.at[pl.ds(base, chunk)], rows_v.at[b], osem[b]
      ).wait()

    for b in range(nbuf):
      start_gather(b, b)

    def group_body(g, carry):
      c0 = g * nbuf
      for b in range(nbuf):
        wait_gather(b)
        start_out(b, c0 + b)
      for b in range(nbuf):
        wait_out(b)
        start_gather(b, c0 + nbuf + b)
      return carry

    lax.fori_loop(0, n_groups - 1, group_body, 0)

    c0 = (n_groups - 1) * nbuf
    for b in range(nbuf):
      wait_gather(b)
      start_out(b, c0 + b)
    for b in range(nbuf):
      wait_out(b)

  return gather_kernel


def kernel(inp, table):
  b, s = inp.shape
  vocab, dim = table.shape
  n = b * s
  inp = inp.astype(jnp.int32)

  per_w = (vocab // 128) // _NW
  fast = (
      b == 128 * _NW
      and s % 8 == 0
      and dim == 32
      and vocab % 16 == 0
      and per_w >= 2
      and per_w % 2 == 0
  )
  if fast:
    n_bt = b // 128
    nblk = vocab // 128
    rem = vocab - nblk * 128
    if rem:
      tail = table[nblk * 128 :].reshape(rem * dim)
    else:
      tail = jnp.zeros((128,), jnp.float32)
    table_lin = _make_table_linear(vocab, dim)(table.T, tail)
    table_rm = table_lin.reshape(vocab, dim)
    inp4 = inp.T.reshape(s // 8, 8, n_bt, 128).transpose(0, 2, 1, 3)
    out4 = _make_gather_pack(vocab, s, n_bt, dim)(table_rm, inp4)
    out5 = out4.reshape(s, dim // 8, n_bt, 8, 128)
    return out5.transpose(2, 4, 0, 1, 3).reshape(b, s, dim)

  flat = inp.reshape(n)
  chunk, nbuf = 640, 5
  pad = (-n) % (_NW * chunk * nbuf)
  if pad:
    flat = jnp.concatenate([flat, jnp.zeros((pad,), jnp.int32)])
  rows = _make_gather(vocab, dim, n + pad, chunk, nbuf)(table, flat)
  if pad:
    rows = rows[:n]
  return rows.reshape(b, s, dim)
